# baseline (speedup 1.0000x reference)
import jax
import jax.numpy as jnp
from jax.experimental import pallas as pl


_BLOCK_ROWS = 16


def _split_kernel(x_ref, season_ref, trend_ref):
    i = pl.program_id(0)
    xb = x_ref[...]
    row = jax.lax.broadcasted_iota(jnp.int32, xb.shape, 0) + i * _BLOCK_ROWS
    is_row0 = row == 0
    season_ref[...] = jnp.where(is_row0, 0.0, xb)
    trend_ref[...] = jnp.where(is_row0, xb, 0.0)


def kernel(x):
    rows, cols = x.shape
    grid = (rows // _BLOCK_ROWS,)
    spec = pl.BlockSpec((_BLOCK_ROWS, cols), lambda i: (i, 0))
    season, trend = pl.pallas_call(
        _split_kernel,
        grid=grid,
        in_specs=[spec],
        out_specs=[spec, spec],
        out_shape=[
            jax.ShapeDtypeStruct((rows, cols), x.dtype),
            jax.ShapeDtypeStruct((rows, cols), x.dtype),
        ],
    )(x)
    return (season, trend)


# speedup vs baseline: 1.1167x; 1.1167x over previous
import jax
import jax.numpy as jnp
from jax.experimental import pallas as pl
from jax.experimental.pallas import tpu as pltpu


_BLOCK_ROWS = 32


def _split_kernel(x_ref, season_ref, trend_ref):
    i = pl.program_id(0)
    xb = x_ref[...]
    row = jax.lax.broadcasted_iota(jnp.int32, xb.shape, 0) + i * _BLOCK_ROWS
    is_row0 = row == 0
    season_ref[...] = jnp.where(is_row0, 0.0, xb)
    trend_ref[...] = jnp.where(is_row0, xb, 0.0)


def kernel(x):
    rows, cols = x.shape
    grid = (rows // _BLOCK_ROWS,)
    spec = pl.BlockSpec((_BLOCK_ROWS, cols), lambda i: (i, 0))
    season, trend = pl.pallas_call(
        _split_kernel,
        grid=grid,
        in_specs=[spec],
        out_specs=[spec, spec],
        out_shape=[
            jax.ShapeDtypeStruct((rows, cols), x.dtype),
            jax.ShapeDtypeStruct((rows, cols), x.dtype),
        ],
        compiler_params=pltpu.CompilerParams(
            dimension_semantics=("parallel",),
        ),
    )(x)
    return (season, trend)
